# Initial kernel scaffold; baseline (speedup 1.0000x reference)
import jax
import jax.numpy as jnp
from jax.experimental import pallas as pl


def kernel(x, E0, E1, W1, b1, W2, b2, W3, b3):
    raise NotImplementedError("write your pallas kernel here")


# speedup vs baseline: 1.1509x; 1.1509x over previous
import functools

import jax
import jax.numpy as jnp
from jax import lax
from jax.experimental import pallas as pl
from jax.experimental.pallas import tpu as pltpu
from jax.experimental.pallas import tpu_sc as plsc

NC = 2
NS = 16
NW = NC * NS
CHUNK = 128


def _gather_body(nchunk, x0_hbm, x1_hbm, e0_hbm, e1_hbm, out0_hbm, out1_hbm,
                 idx0_v, idx1_v, rows0_v, rows1_v, sem):
    wid = lax.axis_index("s") * NC + lax.axis_index("c")
    base = wid * nchunk
    pltpu.sync_copy(x0_hbm.at[pl.ds(base, nchunk)], idx0_v)
    pltpu.sync_copy(x1_hbm.at[pl.ds(base, nchunk)], idx1_v)
    copies = []
    for j in range(nchunk):
        copies.append(pltpu.make_async_copy(
            e0_hbm.at[idx0_v.at[j]], rows0_v.at[j], sem))
        copies.append(pltpu.make_async_copy(
            e1_hbm.at[idx1_v.at[j]], rows1_v.at[j], sem))
    for c in copies:
        c.start()
    for c in copies:
        c.wait()
    pltpu.sync_copy(rows0_v, out0_hbm.at[pl.ds(base, nchunk)])
    pltpu.sync_copy(rows1_v, out1_hbm.at[pl.ds(base, nchunk)])


@functools.partial(jax.jit, static_argnums=(4, 5))
def _sc_gather(x0, x1, e0, e1, b, d):
    nchunk = b // (NW * CHUNK)
    mesh = plsc.VectorSubcoreMesh(core_axis_name="c", subcore_axis_name="s")
    fn = pl.kernel(
        functools.partial(_gather_body, nchunk),
        out_type=(
            jax.ShapeDtypeStruct((NW * nchunk, CHUNK, d), jnp.float32),
            jax.ShapeDtypeStruct((NW * nchunk, CHUNK, d), jnp.float32),
        ),
        mesh=mesh,
        scratch_types=[
            pltpu.VMEM((nchunk, CHUNK), jnp.int32),
            pltpu.VMEM((nchunk, CHUNK), jnp.int32),
            pltpu.VMEM((nchunk, CHUNK, d), jnp.float32),
            pltpu.VMEM((nchunk, CHUNK, d), jnp.float32),
            pltpu.SemaphoreType.DMA,
        ],
        compiler_params=pltpu.CompilerParams(use_tc_tiling_on_sc=False),
    )
    return fn(x0, x1, e0, e1)


def _mlp_body(a0_ref, a1_ref, w1a_ref, w1b_ref, b1_ref, w2_ref, b2_ref,
              w3_ref, b3_ref, o_ref):
    f32 = jnp.float32
    h = (jnp.dot(a0_ref[...], w1a_ref[...], preferred_element_type=f32)
         + jnp.dot(a1_ref[...], w1b_ref[...], preferred_element_type=f32)
         + b1_ref[...])
    h = jnp.maximum(h, 0.0)
    h = jnp.dot(h, w2_ref[...], preferred_element_type=f32) + b2_ref[...]
    h = jnp.maximum(h, 0.0)
    o_ref[...] = jnp.dot(h, w3_ref[...], preferred_element_type=f32) + b3_ref[...]


@jax.jit
def _tc_mlp(e0, e1, w1a, w1b, b1, w2, b2, w3, b3):
    b, d = e0.shape
    n_out = w3.shape[1]
    bm = 2048
    grid = (b // bm,)
    full = lambda shape: pl.BlockSpec(shape, lambda i: (0, 0))
    return pl.pallas_call(
        _mlp_body,
        grid=grid,
        in_specs=[
            pl.BlockSpec((bm, d), lambda i: (i, 0)),
            pl.BlockSpec((bm, d), lambda i: (i, 0)),
            full(w1a.shape),
            full(w1b.shape),
            full(b1.shape),
            full(w2.shape),
            full(b2.shape),
            full(w3.shape),
            full(b3.shape),
        ],
        out_specs=pl.BlockSpec((bm, n_out), lambda i: (i, 0)),
        out_shape=jax.ShapeDtypeStruct((b, n_out), jnp.float32),
    )(e0, e1, w1a, w1b, b1, w2, b2, w3, b3)


def kernel(x, E0, E1, W1, b1, W2, b2, W3, b3):
    b = x.shape[0]
    d = E0.shape[1]
    nchunk = b // (NW * CHUNK)
    x0 = x[:, 0].reshape(NW * nchunk, CHUNK)
    x1 = x[:, 1].reshape(NW * nchunk, CHUNK)
    emb0, emb1 = _sc_gather(x0, x1, E0, E1, b, d)
    emb0 = emb0.reshape(b, d)
    emb1 = emb1.reshape(b, d)
    out = _tc_mlp(emb0, emb1, W1[:d], W1[d:],
                  b1.reshape(1, -1), W2, b2.reshape(1, -1),
                  W3, b3.reshape(1, -1))
    return out
